# baseline (speedup 1.0000x reference)
import functools

import jax
import jax.numpy as jnp
from jax import lax
from jax.experimental import pallas as pl
from jax.experimental.pallas import tpu as pltpu
from jax.experimental.pallas import tpu_sc as plsc

D_MODEL = 64
NB = 256
SEQ = 50
BATCH = 16384
SCALE = 64.0
LANES = 16


@jax.jit
def _run(xT, lut):
    info = plsc.get_sparse_core_info()
    nc = info.num_cores
    n_workers = nc * info.num_subcores
    chunks_per_s = BATCH // NB
    total_chunks = SEQ * chunks_per_s
    cpw = total_chunks // n_workers
    mesh = plsc.VectorSubcoreMesh(core_axis_name="c", subcore_axis_name="s")

    @functools.partial(
        pl.kernel,
        mesh=mesh,
        out_type=jax.ShapeDtypeStruct((SEQ, D_MODEL, BATCH), jnp.float32),
        scratch_types=[
            pltpu.VMEM((NB,), jnp.int32),
            pltpu.VMEM((NB, D_MODEL), jnp.float32),
            pltpu.VMEM((D_MODEL, NB), jnp.float32),
            pltpu.SemaphoreType.DMA,
        ],
        compiler_params=pltpu.CompilerParams(
            use_tc_tiling_on_sc=False, needs_layout_passes=False
        ),
    )
    def k(xT_hbm, lut_hbm, out_hbm, idx_v, gbuf, tbuf, sem):
        wid = lax.axis_index("s") * nc + lax.axis_index("c")
        c0 = wid * cpw
        lane = lax.iota(jnp.int32, LANES)

        def chunk(ci, carry):
            c = c0 + ci
            s = c // chunks_per_s
            b0 = (c % chunks_per_s) * NB
            pltpu.sync_copy(xT_hbm.at[s, pl.ds(b0, NB)], idx_v)
            for g in range(NB // 128):
                pltpu.async_copy(
                    lut_hbm.at[idx_v.at[pl.ds(g * 128, 128)]],
                    gbuf.at[pl.ds(g * 128, 128)],
                    sem,
                )
            for g in range(NB // 128):
                pltpu.make_async_copy(
                    lut_hbm.at[idx_v.at[pl.ds(g * 128, 128)]],
                    gbuf.at[pl.ds(g * 128, 128)],
                    sem,
                ).wait()

            def drow(d, carry2):
                cols = jnp.broadcast_to(d, (LANES,)).astype(jnp.int32)
                for g in range(NB // LANES):
                    rows = lane + (g * LANES)
                    v = plsc.load_gather(gbuf, [rows, cols]) * SCALE
                    tbuf[d, pl.ds(g * LANES, LANES)] = v
                return carry2

            lax.fori_loop(0, D_MODEL, drow, 0)
            pltpu.sync_copy(tbuf, out_hbm.at[s, :, pl.ds(b0, NB)])
            return carry

        lax.fori_loop(0, cpw, chunk, 0)

    return k(xT, lut)


def kernel(x, lut):
    xT = jnp.transpose(x).astype(jnp.int32)
    oT = _run(xT, lut)
    return jnp.transpose(oT, (2, 0, 1))


# speedup vs baseline: 1.1159x; 1.1159x over previous
import functools

import jax
import jax.numpy as jnp
from jax import lax
from jax.experimental import pallas as pl
from jax.experimental.pallas import tpu as pltpu
from jax.experimental.pallas import tpu_sc as plsc

D_MODEL = 64
NB = 256
SEQ = 50
BATCH = 16384
SCALE = 64.0
LANES = 16
NBUF = 2


@jax.jit
def _run(xidx, lut):
    info = plsc.get_sparse_core_info()
    nc = info.num_cores
    n_workers = nc * info.num_subcores
    chunks_per_s = BATCH // NB
    total_chunks = SEQ * chunks_per_s
    cpw = total_chunks // n_workers
    n_laps = cpw // NBUF
    mesh = plsc.VectorSubcoreMesh(core_axis_name="c", subcore_axis_name="s")

    @functools.partial(
        pl.kernel,
        mesh=mesh,
        out_type=jax.ShapeDtypeStruct((SEQ, D_MODEL, BATCH), jnp.float32),
        scratch_types=[
            pltpu.VMEM((cpw, NB // 128, 128), jnp.int32),
            pltpu.VMEM((NBUF, NB, D_MODEL), jnp.float32),
            pltpu.VMEM((NBUF, D_MODEL, NB), jnp.float32),
            pltpu.SemaphoreType.DMA((NBUF,)),
            pltpu.SemaphoreType.DMA((NBUF,)),
        ],
        compiler_params=pltpu.CompilerParams(
            use_tc_tiling_on_sc=False, needs_layout_passes=False
        ),
    )
    def k(xidx_hbm, lut_hbm, out_hbm, idx_v, gbuf, tbuf, gsem, osem):
        wid = lax.axis_index("s") * nc + lax.axis_index("c")
        c0 = wid * cpw
        pltpu.sync_copy(xidx_hbm.at[pl.ds(c0, cpw)], idx_v)
        lane = lax.iota(jnp.int32, LANES)
        rows = [lane + g * LANES for g in range(NB // LANES)]

        def start_gathers(ci, b):
            for g in range(NB // 128):
                pltpu.async_copy(
                    lut_hbm.at[idx_v.at[ci, g]],
                    gbuf.at[b, pl.ds(g * 128, 128)],
                    gsem.at[b],
                )

        def wait_gathers(ci, b):
            for g in range(NB // 128):
                pltpu.make_async_copy(
                    lut_hbm.at[idx_v.at[ci, g]],
                    gbuf.at[b, pl.ds(g * 128, 128)],
                    gsem.at[b],
                ).wait()

        def out_slice(ci):
            c = c0 + ci
            s = c // chunks_per_s
            b0 = (c % chunks_per_s) * NB
            return out_hbm.at[s, :, pl.ds(b0, NB)]

        for b in range(NBUF):
            start_gathers(b, b)

        def lap(t, carry):
            for b in range(NBUF):
                ci = t * NBUF + b
                wait_gathers(ci, b)

                @pl.when(t > 0)
                def _wait_out():
                    pltpu.make_async_copy(
                        tbuf.at[b], out_slice(ci - NBUF), osem.at[b]
                    ).wait()

                def drow(d, carry2):
                    cols = jnp.broadcast_to(d, (LANES,)).astype(jnp.int32)
                    for g in range(NB // LANES):
                        v = plsc.load_gather(gbuf.at[b], [rows[g], cols]) * SCALE
                        tbuf[b, d, pl.ds(g * LANES, LANES)] = v
                    return carry2

                lax.fori_loop(0, D_MODEL, drow, 0)
                pltpu.async_copy(tbuf.at[b], out_slice(ci), osem.at[b])

                @pl.when(t < n_laps - 1)
                def _refill():
                    start_gathers(ci + NBUF, b)

            return carry

        lax.fori_loop(0, n_laps, lap, 0)
        for b in range(NBUF):
            pltpu.make_async_copy(
                tbuf.at[b], out_slice(cpw - NBUF + b), osem.at[b]
            ).wait()

    return k(xidx, lut)


def kernel(x, lut):
    xidx = jnp.transpose(x).astype(jnp.int32).reshape(SEQ * BATCH // NB, NB // 128, 128)
    oT = _run(xidx, lut)
    return jnp.transpose(oT, (2, 0, 1))


# speedup vs baseline: 1.7948x; 1.6085x over previous
import functools

import jax
import jax.numpy as jnp
from jax import lax
from jax.experimental import pallas as pl
from jax.experimental.pallas import tpu as pltpu
from jax.experimental.pallas import tpu_sc as plsc

D_MODEL = 64
NB = 256
SEQ = 50
BATCH = 16384
SCALE = 64.0
LANES = 16
NBUF = 2


@jax.jit
def _run(xidx, lut):
    info = plsc.get_sparse_core_info()
    nc = info.num_cores
    n_workers = nc * info.num_subcores
    chunks_per_s = BATCH // NB
    total_chunks = SEQ * chunks_per_s
    cpw = total_chunks // n_workers
    n_laps = cpw // NBUF
    mesh = plsc.VectorSubcoreMesh(core_axis_name="c", subcore_axis_name="s")

    @functools.partial(
        pl.kernel,
        mesh=mesh,
        out_type=jax.ShapeDtypeStruct((SEQ, D_MODEL, BATCH), jnp.float32),
        scratch_types=[
            pltpu.VMEM((cpw, NB // 128, 128), jnp.int32),
            pltpu.VMEM((NBUF, NB, D_MODEL), jnp.float32),
            pltpu.VMEM((NBUF, D_MODEL, NB + 1), jnp.float32),
            pltpu.SemaphoreType.DMA((NBUF,)),
            pltpu.SemaphoreType.DMA((NBUF,)),
        ],
        compiler_params=pltpu.CompilerParams(
            use_tc_tiling_on_sc=False, needs_layout_passes=False
        ),
    )
    def k(xidx_hbm, lut_hbm, out_hbm, idx_v, gbuf, tbuf, gsem, osem):
        wid = lax.axis_index("s") * nc + lax.axis_index("c")
        c0 = wid * cpw
        pltpu.sync_copy(xidx_hbm.at[pl.ds(c0, cpw)], idx_v)
        lane = lax.iota(jnp.int32, LANES)
        rows = [lane + g * LANES for g in range(NB // LANES)]

        def start_gathers(ci, b):
            for g in range(NB // 128):
                pltpu.async_copy(
                    lut_hbm.at[idx_v.at[ci, g]],
                    gbuf.at[b, pl.ds(g * 128, 128)],
                    gsem.at[b],
                )

        def wait_gathers(ci, b):
            for g in range(NB // 128):
                pltpu.make_async_copy(
                    lut_hbm.at[idx_v.at[ci, g]],
                    gbuf.at[b, pl.ds(g * 128, 128)],
                    gsem.at[b],
                ).wait()

        def out_slice(ci):
            c = c0 + ci
            s = c // chunks_per_s
            b0 = (c % chunks_per_s) * NB
            return out_hbm.at[s, :, pl.ds(b0, NB)]

        for b in range(NBUF):
            start_gathers(b, b)

        def lap(t, carry):
            for b in range(NBUF):
                ci = t * NBUF + b
                wait_gathers(ci, b)

                @pl.when(t > 0)
                def _wait_out():
                    pltpu.make_async_copy(
                        tbuf.at[b, :, pl.ds(0, NB)], out_slice(ci - NBUF), osem.at[b]
                    ).wait()

                def trow(r, carry2):
                    cols = jnp.broadcast_to(r, (LANES,)).astype(jnp.int32)
                    for g in range(D_MODEL // LANES):
                        v = gbuf[b, r, pl.ds(g * LANES, LANES)] * SCALE
                        plsc.store_scatter(tbuf.at[b], [rows[g], cols], v)
                    return carry2

                lax.fori_loop(0, NB, trow, 0)
                pltpu.async_copy(
                    tbuf.at[b, :, pl.ds(0, NB)], out_slice(ci), osem.at[b]
                )

                @pl.when(t < n_laps - 1)
                def _refill():
                    start_gathers(ci + NBUF, b)

            return carry

        lax.fori_loop(0, n_laps, lap, 0)
        for b in range(NBUF):
            pltpu.make_async_copy(
                tbuf.at[b, :, pl.ds(0, NB)], out_slice(cpw - NBUF + b), osem.at[b]
            ).wait()

    return k(xidx, lut)


def kernel(x, lut):
    xidx = jnp.transpose(x).astype(jnp.int32).reshape(SEQ * BATCH // NB, NB // 128, 128)
    oT = _run(xidx, lut)
    return jnp.transpose(oT, (2, 0, 1))


# speedup vs baseline: 1.8218x; 1.0150x over previous
import functools

import jax
import jax.numpy as jnp
from jax import lax
from jax.experimental import pallas as pl
from jax.experimental.pallas import tpu as pltpu
from jax.experimental.pallas import tpu_sc as plsc

D_MODEL = 64
NB = 256
SEQ = 50
BATCH = 16384
SCALE = 64.0
LANES = 16
NBUF = 2


@jax.jit
def _run(xidx, lut):
    info = plsc.get_sparse_core_info()
    nc = info.num_cores
    n_workers = nc * info.num_subcores
    chunks_per_s = BATCH // NB
    total_chunks = SEQ * chunks_per_s
    cpw = total_chunks // n_workers
    n_laps = cpw // NBUF
    mesh = plsc.VectorSubcoreMesh(core_axis_name="c", subcore_axis_name="s")

    @functools.partial(
        pl.kernel,
        mesh=mesh,
        out_type=jax.ShapeDtypeStruct((SEQ, D_MODEL, BATCH), jnp.float32),
        scratch_types=[
            pltpu.VMEM((cpw, NB // 128, 128), jnp.int32),
            pltpu.VMEM((NBUF, NB, D_MODEL), jnp.float32),
            pltpu.VMEM((NBUF, D_MODEL, NB + 8), jnp.float32),
            pltpu.SemaphoreType.DMA((NBUF,)),
            pltpu.SemaphoreType.DMA((NBUF,)),
        ],
        compiler_params=pltpu.CompilerParams(
            use_tc_tiling_on_sc=False, needs_layout_passes=False
        ),
    )
    def k(xidx_hbm, lut_hbm, out_hbm, idx_v, gbuf, tbuf, gsem, osem):
        wid = lax.axis_index("s") * nc + lax.axis_index("c")
        c0 = wid * cpw
        pltpu.sync_copy(xidx_hbm.at[pl.ds(c0, cpw)], idx_v)
        lane = lax.iota(jnp.int32, LANES)
        rows = [lane + g * LANES for g in range(NB // LANES)]

        def start_gathers(ci, b):
            for g in range(NB // 128):
                pltpu.async_copy(
                    lut_hbm.at[idx_v.at[ci, g]],
                    gbuf.at[b, pl.ds(g * 128, 128)],
                    gsem.at[b],
                )

        def wait_gathers(ci, b):
            for g in range(NB // 128):
                pltpu.make_async_copy(
                    lut_hbm.at[idx_v.at[ci, g]],
                    gbuf.at[b, pl.ds(g * 128, 128)],
                    gsem.at[b],
                ).wait()

        def out_slice(ci):
            c = c0 + ci
            s = c // chunks_per_s
            b0 = (c % chunks_per_s) * NB
            return out_hbm.at[s, :, pl.ds(b0, NB)]

        for b in range(NBUF):
            start_gathers(b, b)

        def lap(t, carry):
            for b in range(NBUF):
                ci = t * NBUF + b
                wait_gathers(ci, b)

                @pl.when(t > 0)
                def _wait_out():
                    pltpu.make_async_copy(
                        tbuf.at[b, :, pl.ds(0, NB)], out_slice(ci - NBUF), osem.at[b]
                    ).wait()

                def trow(r4, carry2):
                    r0 = r4 * 4
                    for rr in range(4):
                        r = r0 + rr
                        cols = jnp.broadcast_to(r, (LANES,)).astype(jnp.int32)
                        for g in range(D_MODEL // LANES):
                            v = gbuf[b, r, pl.ds(g * LANES, LANES)] * SCALE
                            plsc.store_scatter(tbuf.at[b], [rows[g], cols], v)
                    return carry2

                lax.fori_loop(0, NB // 4, trow, 0)
                pltpu.async_copy(
                    tbuf.at[b, :, pl.ds(0, NB)], out_slice(ci), osem.at[b]
                )

                @pl.when(t < n_laps - 1)
                def _refill():
                    start_gathers(ci + NBUF, b)

            return carry

        lax.fori_loop(0, n_laps, lap, 0)
        for b in range(NBUF):
            pltpu.make_async_copy(
                tbuf.at[b, :, pl.ds(0, NB)], out_slice(cpw - NBUF + b), osem.at[b]
            ).wait()

    return k(xidx, lut)


def kernel(x, lut):
    xidx = jnp.transpose(x).astype(jnp.int32).reshape(SEQ * BATCH // NB, NB // 128, 128)
    oT = _run(xidx, lut)
    return jnp.transpose(oT, (2, 0, 1))
